# Initial kernel scaffold; baseline (speedup 1.0000x reference)
import jax
import jax.numpy as jnp
from jax.experimental import pallas as pl


def kernel(x, edge_index, W1, al1, ar1, W2, al2, ar2):
    raise NotImplementedError("write your pallas kernel here")


# speedup vs baseline: 10.8680x; 10.8680x over previous
import functools

import jax
import jax.numpy as jnp
from jax import lax
from jax.experimental import pallas as pl
from jax.experimental.pallas import tpu as pltpu
from jax.experimental.pallas import tpu_sc as plsc

N = 10000
E = 320000
IN_FEATS = 128
N_HIDDEN = 64
HEADS = (8, 1)
N_CLASSES = 41
NEG_SLOPE = 0.2

NW = 32
K = 128
E_PAD = 323584
N_JUNK = 112
NA = N + N_JUNK
RPT = NA // 16
NB = 25
BN = N // NB
DP = 128


def _tc_layer1(x, W1, al1, ar1):
    def body(x_ref, w_ref, al_ref, ar_ref, tab_ref, el_ref, er_ref):
        h = jnp.dot(x_ref[...], w_ref[...], preferred_element_type=jnp.float32)
        hr = h.reshape(BN, 8, N_HIDDEN)
        el = jnp.sum(hr * al_ref[...][None, :, :], axis=-1)
        er = jnp.sum(hr * ar_ref[...][None, :, :], axis=-1)
        el_ref[...] = el
        er_ref[...] = er
        ht = jnp.transpose(hr, (1, 0, 2))
        ones = jnp.ones((8, BN, 1), jnp.float32)
        zeros = jnp.zeros((8, BN, DP - N_HIDDEN - 1), jnp.float32)
        tab_ref[...] = jnp.concatenate([ht, ones, zeros], axis=-1)

    return pl.pallas_call(
        body,
        grid=(NB,),
        in_specs=[
            pl.BlockSpec((BN, IN_FEATS), lambda i: (i, 0)),
            pl.BlockSpec((IN_FEATS, 8 * N_HIDDEN), lambda i: (0, 0)),
            pl.BlockSpec((8, N_HIDDEN), lambda i: (0, 0)),
            pl.BlockSpec((8, N_HIDDEN), lambda i: (0, 0)),
        ],
        out_specs=[
            pl.BlockSpec((8, BN, DP), lambda i: (0, i, 0)),
            pl.BlockSpec((BN, 8), lambda i: (i, 0)),
            pl.BlockSpec((BN, 8), lambda i: (i, 0)),
        ],
        out_shape=[
            jax.ShapeDtypeStruct((8, N, DP), jnp.float32),
            jax.ShapeDtypeStruct((N, 8), jnp.float32),
            jax.ShapeDtypeStruct((N, 8), jnp.float32),
        ],
    )(x, W1, al1, ar1)


def _tc_layer2(acc1, W2, al2, ar2):
    def body(a_ref, w_ref, al_ref, ar_ref, tab_ref, el_ref, er_ref):
        a = a_ref[...][0] + a_ref[...][1]
        s = a[:, :, N_HIDDEN:N_HIDDEN + 1]
        o = a[:, :, 0:64] / (s + 1e-9)
        o = jnp.where(o > 0, o, jnp.exp(o) - 1.0)
        h1 = jnp.transpose(o, (1, 0, 2)).reshape(BN, 8 * N_HIDDEN)
        h2 = jnp.dot(h1, w_ref[...], preferred_element_type=jnp.float32)
        el = jnp.sum(h2 * al_ref[...], axis=-1)
        er = jnp.sum(h2 * ar_ref[...], axis=-1)
        el_ref[...] = el[:, None]
        er_ref[...] = er[:, None]
        ones = jnp.ones((BN, 1), jnp.float32)
        zeros = jnp.zeros((BN, DP - N_CLASSES - 1), jnp.float32)
        tab_ref[...] = jnp.concatenate([h2, ones, zeros], axis=-1)

    return pl.pallas_call(
        body,
        grid=(NB,),
        in_specs=[
            pl.BlockSpec((2, 8, BN, DP), lambda i: (0, 0, i, 0)),
            pl.BlockSpec((8 * N_HIDDEN, N_CLASSES), lambda i: (0, 0)),
            pl.BlockSpec((1, N_CLASSES), lambda i: (0, 0)),
            pl.BlockSpec((1, N_CLASSES), lambda i: (0, 0)),
        ],
        out_specs=[
            pl.BlockSpec((BN, DP), lambda i: (i, 0)),
            pl.BlockSpec((BN, 1), lambda i: (i, 0)),
            pl.BlockSpec((BN, 1), lambda i: (i, 0)),
        ],
        out_shape=[
            jax.ShapeDtypeStruct((N, DP), jnp.float32),
            jax.ShapeDtypeStruct((N, 1), jnp.float32),
            jax.ShapeDtypeStruct((N, 1), jnp.float32),
        ],
    )(acc1, W2, al2, ar2)


def _tc_final(acc2):
    def body(a_ref, out_ref):
        a = a_ref[...][0] + a_ref[...][1]
        out_ref[...] = a[:, 0:N_CLASSES] / (a[:, N_CLASSES:N_CLASSES + 1] + 1e-9)

    return pl.pallas_call(
        body,
        grid=(NB,),
        in_specs=[pl.BlockSpec((2, BN, DP), lambda i: (0, i, 0))],
        out_specs=pl.BlockSpec((BN, N_CLASSES), lambda i: (i, 0)),
        out_shape=jax.ShapeDtypeStruct((N, N_CLASSES), jnp.float32),
    )(acc2)


def _make_sc_edge(num_heads, nsq):
    ew = E_PAD // NW
    nch = ew // K

    mesh = plsc.VectorSubcoreMesh(core_axis_name="c", subcore_axis_name="s")

    @functools.partial(
        pl.kernel,
        mesh=mesh,
        compiler_params=pltpu.CompilerParams(needs_layout_passes=False),
        out_type=jax.ShapeDtypeStruct((2 * num_heads * NA, DP), jnp.float32),
        scratch_types=[
            pltpu.VMEM((N,), jnp.float32),
            pltpu.VMEM((N,), jnp.float32),
            pltpu.VMEM((K,), jnp.int32),
            pltpu.VMEM((K,), jnp.int32),
            pltpu.VMEM((K,), jnp.float32),
            pltpu.VMEM((K, DP), jnp.float32),
            pltpu.VMEM((64, DP), jnp.float32),
            pltpu.VMEM_SHARED((NA, DP), jnp.float32),
            pltpu.SemaphoreType.DMA,
        ],
    )
    def sc_kernel(tab_hbm, el_hbm, er_hbm, src_hbm, dst_hbm, out_hbm,
                  el_v, er_v, src_v, dst_v, w_v, rows_v, z_v, acc, sem):
        c = lax.axis_index("c")
        s = lax.axis_index("s")
        wid = s * 2 + c
        row0 = s * RPT
        zvec = jnp.zeros((16,), jnp.float32)
        for r in range(64):
            for q in range(DP // 16):
                z_v[r, pl.ds(16 * q, 16)] = zvec
        cols = [lax.iota(jnp.int32, 16) + 16 * q for q in range(nsq)]

        def zero_acc():
            for kk in range(RPT // 64):
                pltpu.sync_copy(z_v, acc.at[pl.ds(row0 + 64 * kk, 64)])
            rem = RPT % 64
            if rem:
                pltpu.sync_copy(z_v.at[pl.ds(0, rem)],
                                acc.at[pl.ds(row0 + (RPT // 64) * 64, rem)])

        zero_acc()
        plsc.subcore_barrier()

        for h in range(num_heads):
            pltpu.sync_copy(el_hbm.at[pl.ds(h * N, N)], el_v)
            pltpu.sync_copy(er_hbm.at[pl.ds(h * N, N)], er_v)

            def chunk_body(t, _):
                base = wid * ew + t * K
                pltpu.sync_copy(src_hbm.at[pl.ds(base, K)], src_v)
                pltpu.sync_copy(dst_hbm.at[pl.ds(base, K)], dst_v)

                def jbody(j, _):
                    s16 = src_v[pl.ds(j * 16, 16)]
                    d16 = dst_v[pl.ds(j * 16, 16)]
                    dc = jnp.minimum(d16, N - 1)
                    e16 = (plsc.load_gather(el_v, [s16])
                           + plsc.load_gather(er_v, [dc]))
                    e16 = jnp.where(e16 > 0, e16, NEG_SLOPE * e16)
                    w_v[pl.ds(j * 16, 16)] = jnp.exp(e16)
                    if num_heads > 1:
                        src_v[pl.ds(j * 16, 16)] = s16 + h * N
                    return 0

                lax.fori_loop(0, K // 16, jbody, 0)
                pltpu.async_copy(tab_hbm.at[src_v], rows_v, sem).wait()

                def ibody(i, _):
                    ri = jnp.full((16,), i, jnp.int32)
                    wv = plsc.load_gather(w_v, [ri])
                    for q in range(nsq):
                        vals = plsc.load_gather(rows_v, [ri, cols[q]])
                        plsc.store_scatter(rows_v, [ri, cols[q]], vals * wv)
                    return 0

                lax.fori_loop(0, K, ibody, 0)
                pltpu.sync_copy(rows_v, acc.at[dst_v], add=True)
                return 0

            lax.fori_loop(0, nch, chunk_body, 0)
            plsc.subcore_barrier()
            dbase = (c * num_heads + h) * NA + row0
            pltpu.sync_copy(acc.at[pl.ds(row0, RPT)],
                            out_hbm.at[pl.ds(dbase, RPT)])
            if h + 1 < num_heads:
                zero_acc()
                plsc.subcore_barrier()

    return sc_kernel


_sc_edge_l1 = _make_sc_edge(8, 5)
_sc_edge_l2 = _make_sc_edge(1, 3)


def kernel(x, edge_index, W1, al1, ar1, W2, al2, ar2):
    npad = E_PAD - E
    pad_src = jnp.arange(npad, dtype=jnp.int32) % N
    pad_dst = N + jnp.arange(npad, dtype=jnp.int32) % N_JUNK
    src = jnp.concatenate([edge_index[0], pad_src])
    dst = jnp.concatenate([edge_index[1], pad_dst])

    tab1, el1, er1 = _tc_layer1(x, W1, al1, ar1)
    acc1 = _sc_edge_l1(tab1.reshape(8 * N, DP), el1.T.reshape(8 * N),
                       er1.T.reshape(8 * N), src, dst)
    tab2, el2, er2 = _tc_layer2(acc1.reshape(2, 8, NA, DP), W2, al2, ar2)
    acc2 = _sc_edge_l2(tab2, el2.reshape(N), er2.reshape(N), src, dst)
    return _tc_final(acc2.reshape(2, NA, DP))


# speedup vs baseline: 13.1544x; 1.2104x over previous
import functools

import jax
import jax.numpy as jnp
from jax import lax
from jax.experimental import pallas as pl
from jax.experimental.pallas import tpu as pltpu
from jax.experimental.pallas import tpu_sc as plsc

N = 10000
E = 320000
IN_FEATS = 128
N_HIDDEN = 64
HEADS = (8, 1)
N_CLASSES = 41
NEG_SLOPE = 0.2

NW = 32
K = 96
E_PAD = 331776
N_JUNK = 112
NA = N + N_JUNK
RPT = NA // 16
NB = 25
BN = N // NB
DP = 128


def _tc_layer1(x, W1, al1, ar1):
    def body(x_ref, w_ref, al_ref, ar_ref, tab_ref, el_ref, er_ref):
        h = jnp.dot(x_ref[...], w_ref[...], preferred_element_type=jnp.float32)
        hr = h.reshape(BN, 8, N_HIDDEN)
        el = jnp.sum(hr * al_ref[...][None, :, :], axis=-1)
        er = jnp.sum(hr * ar_ref[...][None, :, :], axis=-1)
        el_ref[...] = el
        er_ref[...] = er
        ht = jnp.transpose(hr, (1, 0, 2))
        ones = jnp.ones((8, BN, 1), jnp.float32)
        zeros = jnp.zeros((8, BN, DP - N_HIDDEN - 1), jnp.float32)
        tab_ref[...] = jnp.concatenate([ht, ones, zeros], axis=-1)

    return pl.pallas_call(
        body,
        grid=(NB,),
        in_specs=[
            pl.BlockSpec((BN, IN_FEATS), lambda i: (i, 0)),
            pl.BlockSpec((IN_FEATS, 8 * N_HIDDEN), lambda i: (0, 0)),
            pl.BlockSpec((8, N_HIDDEN), lambda i: (0, 0)),
            pl.BlockSpec((8, N_HIDDEN), lambda i: (0, 0)),
        ],
        out_specs=[
            pl.BlockSpec((8, BN, DP), lambda i: (0, i, 0)),
            pl.BlockSpec((BN, 8), lambda i: (i, 0)),
            pl.BlockSpec((BN, 8), lambda i: (i, 0)),
        ],
        out_shape=[
            jax.ShapeDtypeStruct((8, N, DP), jnp.float32),
            jax.ShapeDtypeStruct((N, 8), jnp.float32),
            jax.ShapeDtypeStruct((N, 8), jnp.float32),
        ],
    )(x, W1, al1, ar1)


def _tc_layer2(acc1, W2, al2, ar2):
    def body(a_ref, w_ref, al_ref, ar_ref, tab_ref, el_ref, er_ref):
        a = a_ref[...][0] + a_ref[...][1]
        s = a[:, :, N_HIDDEN:N_HIDDEN + 1]
        o = a[:, :, 0:64] / (s + 1e-9)
        o = jnp.where(o > 0, o, jnp.exp(o) - 1.0)
        h1 = jnp.transpose(o, (1, 0, 2)).reshape(BN, 8 * N_HIDDEN)
        h2 = jnp.dot(h1, w_ref[...], preferred_element_type=jnp.float32)
        el = jnp.sum(h2 * al_ref[...], axis=-1)
        er = jnp.sum(h2 * ar_ref[...], axis=-1)
        el_ref[...] = el[:, None]
        er_ref[...] = er[:, None]
        ones = jnp.ones((BN, 1), jnp.float32)
        zeros = jnp.zeros((BN, DP - N_CLASSES - 1), jnp.float32)
        tab_ref[...] = jnp.concatenate([h2, ones, zeros], axis=-1)

    return pl.pallas_call(
        body,
        grid=(NB,),
        in_specs=[
            pl.BlockSpec((2, 8, BN, DP), lambda i: (0, 0, i, 0)),
            pl.BlockSpec((8 * N_HIDDEN, N_CLASSES), lambda i: (0, 0)),
            pl.BlockSpec((1, N_CLASSES), lambda i: (0, 0)),
            pl.BlockSpec((1, N_CLASSES), lambda i: (0, 0)),
        ],
        out_specs=[
            pl.BlockSpec((BN, DP), lambda i: (i, 0)),
            pl.BlockSpec((BN, 1), lambda i: (i, 0)),
            pl.BlockSpec((BN, 1), lambda i: (i, 0)),
        ],
        out_shape=[
            jax.ShapeDtypeStruct((N, DP), jnp.float32),
            jax.ShapeDtypeStruct((N, 1), jnp.float32),
            jax.ShapeDtypeStruct((N, 1), jnp.float32),
        ],
    )(acc1, W2, al2, ar2)


def _tc_final(acc2):
    def body(a_ref, out_ref):
        a = a_ref[...][0] + a_ref[...][1]
        out_ref[...] = a[:, 0:N_CLASSES] / (a[:, N_CLASSES:N_CLASSES + 1] + 1e-9)

    return pl.pallas_call(
        body,
        grid=(NB,),
        in_specs=[pl.BlockSpec((2, BN, DP), lambda i: (0, i, 0))],
        out_specs=pl.BlockSpec((BN, N_CLASSES), lambda i: (i, 0)),
        out_shape=jax.ShapeDtypeStruct((N, N_CLASSES), jnp.float32),
    )(acc2)


def _make_sc_edge(num_heads, nsq):
    ew = E_PAD // NW
    npair = ew // (2 * K)

    mesh = plsc.VectorSubcoreMesh(core_axis_name="c", subcore_axis_name="s")

    @functools.partial(
        pl.kernel,
        mesh=mesh,
        compiler_params=pltpu.CompilerParams(needs_layout_passes=False),
        out_type=jax.ShapeDtypeStruct((2 * num_heads * NA, DP), jnp.float32),
        scratch_types=[
            pltpu.VMEM((N,), jnp.float32),
            pltpu.VMEM((N,), jnp.float32),
            pltpu.VMEM((2 * K,), jnp.int32),
            pltpu.VMEM((2 * K,), jnp.int32),
            pltpu.VMEM((K,), jnp.int32),
            pltpu.VMEM((K,), jnp.int32),
            pltpu.VMEM((K,), jnp.int32),
            pltpu.VMEM((K,), jnp.int32),
            pltpu.VMEM((K,), jnp.float32),
            pltpu.VMEM((K,), jnp.float32),
            pltpu.VMEM((K, DP), jnp.float32),
            pltpu.VMEM((K, DP), jnp.float32),
            pltpu.VMEM_SHARED((NA, DP), jnp.float32),
            pltpu.SemaphoreType.DMA,
            pltpu.SemaphoreType.DMA,
            pltpu.SemaphoreType.DMA,
            pltpu.SemaphoreType.DMA,
        ],
    )
    def sc_kernel(tab_hbm, el_hbm, er_hbm, sd_hbm, out_hbm,
                  el_v, er_v, sd_a, sd_b, gi_a, gi_b, di_a, di_b,
                  w_a, w_b, rows_a, rows_b, acc,
                  gsem_a, gsem_b, csem_a, csem_b):
        c = lax.axis_index("c")
        s = lax.axis_index("s")
        wid = s * 2 + c
        row0 = s * RPT
        zvec = jnp.zeros((16,), jnp.float32)

        def zero_acc():
            for r in range(K):
                for q in range(DP // 16):
                    rows_a[r, pl.ds(16 * q, 16)] = zvec
            for kk in range(RPT // K):
                pltpu.sync_copy(rows_a, acc.at[pl.ds(row0 + K * kk, K)])
            rem = RPT % K
            if rem:
                pltpu.sync_copy(rows_a.at[pl.ds(0, rem)],
                                acc.at[pl.ds(row0 + (RPT // K) * K, rem)])

        zero_acc()
        plsc.subcore_barrier()

        bufs = [
            (sd_a, gi_a, di_a, w_a, rows_a, gsem_a, csem_a),
            (sd_b, gi_b, di_b, w_b, rows_b, gsem_b, csem_b),
        ]

        cols = [lax.iota(jnp.int32, 16) + 16 * q for q in range(nsq)]

        def head_body(h, _):
            hbase = pl.multiple_of(h * N, 8)
            pltpu.sync_copy(el_hbm.at[pl.ds(hbase, N)], el_v)
            pltpu.sync_copy(er_hbm.at[pl.ds(hbase, N)], er_v)
            hoff = jnp.full((16,), h * N, jnp.int32)

            def pair_body(t2, _):
                pbase = (wid * ew + t2 * 2 * K) * 2
                pltpu.sync_copy(sd_hbm.at[pl.ds(pbase, 2 * K)], sd_a)
                pltpu.sync_copy(sd_hbm.at[pl.ds(pbase + 2 * K, 2 * K)], sd_b)

                for sd_v, gi, di, w_v, rows_v, gsem, csem in bufs:
                    @pl.when(t2 > 0)
                    def _():
                        pltpu.make_async_copy(
                            rows_v, acc.at[di], csem).wait()
                    for j in range(K // 16):
                        s16 = sd_v[pl.ds(j * 16, 16)]
                        d16 = sd_v[pl.ds(K + j * 16, 16)]
                        dc = jnp.minimum(d16, N - 1)
                        e16 = (plsc.load_gather(el_v, [s16])
                               + plsc.load_gather(er_v, [dc]))
                        e16 = jnp.where(e16 > 0, e16, NEG_SLOPE * e16)
                        w_v[pl.ds(j * 16, 16)] = jnp.exp(e16)
                        if num_heads > 1:
                            gi[pl.ds(j * 16, 16)] = s16 + hoff
                        else:
                            gi[pl.ds(j * 16, 16)] = s16
                        di[pl.ds(j * 16, 16)] = d16
                    pltpu.async_copy(tab_hbm.at[gi], rows_v, gsem)

                for sd_v, gi, di, w_v, rows_v, gsem, csem in bufs:
                    pltpu.make_async_copy(tab_hbm.at[gi], rows_v, gsem).wait()

                    def scale_body(i, _):
                        ri = jnp.full((16,), i, jnp.int32)
                        wv = plsc.load_gather(w_v, [ri])
                        for q in range(nsq):
                            vals = plsc.load_gather(rows_v, [ri, cols[q]])
                            plsc.store_scatter(rows_v, [ri, cols[q]],
                                               vals * wv)
                        return 0

                    lax.fori_loop(0, K, scale_body, 0)
                    pltpu.async_copy(rows_v, acc.at[di], csem, add=True)
                return 0

            lax.fori_loop(0, npair, pair_body, 0)
            for _, gi, di, w_v, rows_v, gsem, csem in bufs:
                pltpu.make_async_copy(rows_v, acc.at[di], csem).wait()
            plsc.subcore_barrier()
            dbase = pl.multiple_of((c * num_heads + h) * NA + row0, 8)
            pltpu.sync_copy(acc.at[pl.ds(row0, RPT)],
                            out_hbm.at[pl.ds(dbase, RPT)])
            @pl.when(h + 1 < num_heads)
            def _():
                zero_acc()
            plsc.subcore_barrier()
            return 0

        lax.fori_loop(0, num_heads, head_body, 0)

    return sc_kernel


_sc_edge_l1 = _make_sc_edge(8, 5)
_sc_edge_l2 = _make_sc_edge(1, 3)


def kernel(x, edge_index, W1, al1, ar1, W2, al2, ar2):
    npad = E_PAD - E
    pad_src = jnp.arange(npad, dtype=jnp.int32) % N
    pad_dst = N + jnp.arange(npad, dtype=jnp.int32) % N_JUNK
    src = jnp.concatenate([edge_index[0], pad_src])
    dst = jnp.concatenate([edge_index[1], pad_dst])
    sd = jnp.stack([src.reshape(-1, K), dst.reshape(-1, K)], 1).reshape(-1)

    tab1, el1, er1 = _tc_layer1(x, W1, al1, ar1)
    acc1 = _sc_edge_l1(tab1.reshape(8 * N, DP), el1.T.reshape(8 * N),
                       er1.T.reshape(8 * N), sd)
    tab2, el2, er2 = _tc_layer2(acc1.reshape(2, 8, NA, DP), W2, al2, ar2)
    acc2 = _sc_edge_l2(tab2, el2.reshape(N), er2.reshape(N), sd)
    return _tc_final(acc2.reshape(2, NA, DP))


# speedup vs baseline: 16.7425x; 1.2728x over previous
import functools

import jax
import jax.numpy as jnp
from jax import lax
from jax.experimental import pallas as pl
from jax.experimental.pallas import tpu as pltpu
from jax.experimental.pallas import tpu_sc as plsc

N = 10000
E = 320000
IN_FEATS = 128
N_HIDDEN = 64
HEADS = (8, 1)
N_CLASSES = 41
NEG_SLOPE = 0.2

NW = 32
K = 96
E_PAD = 331776
N_JUNK = 112
NA = N + N_JUNK
RPT = NA // 16
NB = 25
BN = N // NB
DP = 128


def _tc_layer1(x, W1, al1, ar1):
    def body(x_ref, w_ref, al_ref, ar_ref, tab_ref, el_ref, er_ref):
        h = jnp.dot(x_ref[...], w_ref[...], preferred_element_type=jnp.float32)
        hr = h.reshape(BN, 8, N_HIDDEN)
        el = jnp.sum(hr * al_ref[...][None, :, :], axis=-1)
        er = jnp.sum(hr * ar_ref[...][None, :, :], axis=-1)
        el_ref[...] = el
        er_ref[...] = er
        ht = jnp.transpose(hr, (1, 0, 2))
        ones = jnp.ones((8, BN, 1), jnp.float32)
        zeros = jnp.zeros((8, BN, DP - N_HIDDEN - 1), jnp.float32)
        tab_ref[...] = jnp.concatenate([ht, ones, zeros], axis=-1)

    return pl.pallas_call(
        body,
        grid=(NB,),
        in_specs=[
            pl.BlockSpec((BN, IN_FEATS), lambda i: (i, 0)),
            pl.BlockSpec((IN_FEATS, 8 * N_HIDDEN), lambda i: (0, 0)),
            pl.BlockSpec((8, N_HIDDEN), lambda i: (0, 0)),
            pl.BlockSpec((8, N_HIDDEN), lambda i: (0, 0)),
        ],
        out_specs=[
            pl.BlockSpec((8, BN, DP), lambda i: (0, i, 0)),
            pl.BlockSpec((BN, 8), lambda i: (i, 0)),
            pl.BlockSpec((BN, 8), lambda i: (i, 0)),
        ],
        out_shape=[
            jax.ShapeDtypeStruct((8, N, DP), jnp.float32),
            jax.ShapeDtypeStruct((N, 8), jnp.float32),
            jax.ShapeDtypeStruct((N, 8), jnp.float32),
        ],
    )(x, W1, al1, ar1)


def _tc_layer2(acc1, W2, al2, ar2):
    def body(a_ref, w_ref, al_ref, ar_ref, tab_ref, el_ref, er_ref):
        a = a_ref[...][0] + a_ref[...][1]
        s = a[:, :, N_HIDDEN:N_HIDDEN + 1]
        o = a[:, :, 0:64] / (s + 1e-9)
        o = jnp.where(o > 0, o, jnp.exp(o) - 1.0)
        h1 = jnp.transpose(o, (1, 0, 2)).reshape(BN, 8 * N_HIDDEN)
        h2 = jnp.dot(h1, w_ref[...], preferred_element_type=jnp.float32)
        el = jnp.sum(h2 * al_ref[...], axis=-1)
        er = jnp.sum(h2 * ar_ref[...], axis=-1)
        el_ref[...] = el[:, None]
        er_ref[...] = er[:, None]
        ones = jnp.ones((BN, 1), jnp.float32)
        zeros = jnp.zeros((BN, DP - N_CLASSES - 1), jnp.float32)
        tab_ref[...] = jnp.concatenate([h2, ones, zeros], axis=-1)

    return pl.pallas_call(
        body,
        grid=(NB,),
        in_specs=[
            pl.BlockSpec((2, 8, BN, DP), lambda i: (0, 0, i, 0)),
            pl.BlockSpec((8 * N_HIDDEN, N_CLASSES), lambda i: (0, 0)),
            pl.BlockSpec((1, N_CLASSES), lambda i: (0, 0)),
            pl.BlockSpec((1, N_CLASSES), lambda i: (0, 0)),
        ],
        out_specs=[
            pl.BlockSpec((BN, DP), lambda i: (i, 0)),
            pl.BlockSpec((BN, 1), lambda i: (i, 0)),
            pl.BlockSpec((BN, 1), lambda i: (i, 0)),
        ],
        out_shape=[
            jax.ShapeDtypeStruct((N, DP), jnp.float32),
            jax.ShapeDtypeStruct((N, 1), jnp.float32),
            jax.ShapeDtypeStruct((N, 1), jnp.float32),
        ],
    )(acc1, W2, al2, ar2)


def _tc_final(acc2):
    def body(a_ref, out_ref):
        a = a_ref[...][0] + a_ref[...][1]
        out_ref[...] = a[:, 0:N_CLASSES] / (a[:, N_CLASSES:N_CLASSES + 1] + 1e-9)

    return pl.pallas_call(
        body,
        grid=(NB,),
        in_specs=[pl.BlockSpec((2, BN, DP), lambda i: (0, i, 0))],
        out_specs=pl.BlockSpec((BN, N_CLASSES), lambda i: (i, 0)),
        out_shape=jax.ShapeDtypeStruct((N, N_CLASSES), jnp.float32),
    )(acc2)


def _make_sc_edge(num_heads, nsq):
    ew = E_PAD // NW
    npair = ew // (2 * K)

    mesh = plsc.VectorSubcoreMesh(core_axis_name="c", subcore_axis_name="s")

    @functools.partial(
        pl.kernel,
        mesh=mesh,
        compiler_params=pltpu.CompilerParams(needs_layout_passes=False),
        out_type=jax.ShapeDtypeStruct((2 * num_heads * NA, DP), jnp.float32),
        scratch_types=[
            pltpu.VMEM((N,), jnp.float32),
            pltpu.VMEM((N,), jnp.float32),
            pltpu.VMEM((2 * K,), jnp.int32),
            pltpu.VMEM((2 * K,), jnp.int32),
            pltpu.VMEM((K,), jnp.int32),
            pltpu.VMEM((K,), jnp.int32),
            pltpu.VMEM((K,), jnp.int32),
            pltpu.VMEM((K,), jnp.int32),
            pltpu.VMEM((K,), jnp.float32),
            pltpu.VMEM((K,), jnp.float32),
            pltpu.VMEM((K, DP), jnp.float32),
            pltpu.VMEM((K, DP), jnp.float32),
            pltpu.VMEM_SHARED((NA, DP), jnp.float32),
            pltpu.SemaphoreType.DMA,
            pltpu.SemaphoreType.DMA,
            pltpu.SemaphoreType.DMA,
            pltpu.SemaphoreType.DMA,
            pltpu.SemaphoreType.DMA,
        ],
    )
    def sc_kernel(tab_hbm, el_hbm, er_hbm, sd_hbm, out_hbm,
                  el_v, er_v, sd_a, sd_b, gi_a, gi_b, di_a, di_b,
                  w_a, w_b, rows_a, rows_b, acc,
                  gsem_a, gsem_b, csem_a, csem_b, ssem):
        c = lax.axis_index("c")
        s = lax.axis_index("s")
        wid = s * 2 + c
        row0 = s * RPT
        zvec = jnp.zeros((16,), jnp.float32)

        def zero_acc():
            for r in range(K):
                for q in range(DP // 16):
                    rows_a[r, pl.ds(16 * q, 16)] = zvec
            for kk in range(RPT // K):
                pltpu.sync_copy(rows_a, acc.at[pl.ds(row0 + K * kk, K)])
            rem = RPT % K
            if rem:
                pltpu.sync_copy(rows_a.at[pl.ds(0, rem)],
                                acc.at[pl.ds(row0 + (RPT // K) * K, rem)])

        zero_acc()
        plsc.subcore_barrier()

        bufs = [
            (sd_a, gi_a, di_a, w_a, rows_a, gsem_a, csem_a),
            (sd_b, gi_b, di_b, w_b, rows_b, gsem_b, csem_b),
        ]

        cols = [lax.iota(jnp.int32, 16) + 16 * q for q in range(nsq)]

        def head_body(h, _):
            hbase = pl.multiple_of(h * N, 8)
            pltpu.sync_copy(el_hbm.at[pl.ds(hbase, N)], el_v)
            pltpu.sync_copy(er_hbm.at[pl.ds(hbase, N)], er_v)
            hoff = jnp.full((16,), h * N, jnp.int32)

            def pair_body(t2, _):
                pbase = (wid * ew + t2 * 2 * K) * 2

                @pl.when(t2 == 0)
                def _():
                    pltpu.sync_copy(sd_hbm.at[pl.ds(pbase, 2 * K)], sd_a)
                    pltpu.sync_copy(sd_hbm.at[pl.ds(pbase + 2 * K, 2 * K)],
                                    sd_b)
                @pl.when(t2 > 0)
                def _():
                    pltpu.make_async_copy(
                        sd_hbm.at[pl.ds(pbase, 2 * K)], sd_a, ssem).wait()
                    pltpu.make_async_copy(
                        sd_hbm.at[pl.ds(pbase + 2 * K, 2 * K)], sd_b,
                        ssem).wait()

                for sd_v, gi, di, w_v, rows_v, gsem, csem in bufs:
                    @pl.when(t2 > 0)
                    def _():
                        pltpu.make_async_copy(
                            rows_v, acc.at[di], csem).wait()
                    for j in range(K // 16):
                        s16 = sd_v[pl.ds(j * 16, 16)]
                        d16 = sd_v[pl.ds(K + j * 16, 16)]
                        dc = jnp.minimum(d16, N - 1)
                        e16 = (plsc.load_gather(el_v, [s16])
                               + plsc.load_gather(er_v, [dc]))
                        e16 = jnp.where(e16 > 0, e16, NEG_SLOPE * e16)
                        w_v[pl.ds(j * 16, 16)] = jnp.exp(e16)
                        if num_heads > 1:
                            gi[pl.ds(j * 16, 16)] = s16 + hoff
                        else:
                            gi[pl.ds(j * 16, 16)] = s16
                        di[pl.ds(j * 16, 16)] = d16
                    pltpu.async_copy(tab_hbm.at[gi], rows_v, gsem)

                @pl.when(t2 + 1 < npair)
                def _():
                    nbase = pbase + 4 * K
                    pltpu.async_copy(sd_hbm.at[pl.ds(nbase, 2 * K)], sd_a,
                                     ssem)
                    pltpu.async_copy(sd_hbm.at[pl.ds(nbase + 2 * K, 2 * K)],
                                     sd_b, ssem)

                for sd_v, gi, di, w_v, rows_v, gsem, csem in bufs:
                    pltpu.make_async_copy(tab_hbm.at[gi], rows_v, gsem).wait()
                    def scale_body(j, _):
                        jb = j * 16
                        for k in range(16):
                            ri = jnp.full((16,), k, jnp.int32) + jb
                            wv = plsc.load_gather(w_v, [ri])
                            for q in range(nsq):
                                vals = plsc.load_gather(rows_v,
                                                        [ri, cols[q]])
                                plsc.store_scatter(rows_v, [ri, cols[q]],
                                                   vals * wv)
                        return 0

                    lax.fori_loop(0, K // 16, scale_body, 0)
                    pltpu.async_copy(rows_v, acc.at[di], csem, add=True)
                return 0

            lax.fori_loop(0, npair, pair_body, 0)
            for _, gi, di, w_v, rows_v, gsem, csem in bufs:
                pltpu.make_async_copy(rows_v, acc.at[di], csem).wait()
            plsc.subcore_barrier()
            dbase = pl.multiple_of((c * num_heads + h) * NA + row0, 8)
            pltpu.sync_copy(acc.at[pl.ds(row0, RPT)],
                            out_hbm.at[pl.ds(dbase, RPT)])
            @pl.when(h + 1 < num_heads)
            def _():
                zero_acc()
            plsc.subcore_barrier()
            return 0

        lax.fori_loop(0, num_heads, head_body, 0)

    return sc_kernel


_sc_edge_l1 = _make_sc_edge(8, 5)
_sc_edge_l2 = _make_sc_edge(1, 3)


def kernel(x, edge_index, W1, al1, ar1, W2, al2, ar2):
    npad = E_PAD - E
    pad_src = jnp.arange(npad, dtype=jnp.int32) % N
    pad_dst = N + jnp.arange(npad, dtype=jnp.int32) % N_JUNK
    src = jnp.concatenate([edge_index[0], pad_src])
    dst = jnp.concatenate([edge_index[1], pad_dst])
    sd = jnp.stack([src.reshape(-1, K), dst.reshape(-1, K)], 1).reshape(-1)

    tab1, el1, er1 = _tc_layer1(x, W1, al1, ar1)
    acc1 = _sc_edge_l1(tab1.reshape(8 * N, DP), el1.T.reshape(8 * N),
                       er1.T.reshape(8 * N), sd)
    tab2, el2, er2 = _tc_layer2(acc1.reshape(2, 8, NA, DP), W2, al2, ar2)
    acc2 = _sc_edge_l2(tab2, el2.reshape(N), er2.reshape(N), sd)
    return _tc_final(acc2.reshape(2, NA, DP))


# speedup vs baseline: 30.7420x; 1.8362x over previous
import functools

import jax
import jax.numpy as jnp
from jax import lax
from jax.experimental import pallas as pl
from jax.experimental.pallas import tpu as pltpu
from jax.experimental.pallas import tpu_sc as plsc

N = 10000
E = 320000
IN_FEATS = 128
N_HIDDEN = 64
HEADS = (8, 1)
N_CLASSES = 41
NEG_SLOPE = 0.2

NW = 32
K = 96
E_PAD = 331776
N_JUNK = 112
NA = N + N_JUNK
RPT = NA // 16
NB = 25
BN = N // NB
DP = 128


def _tc_layer1(x, W1, al1, ar1):
    def body(x_ref, w_ref, al_ref, ar_ref, tab_ref, el_ref, er_ref):
        h = jnp.dot(x_ref[...], w_ref[...], preferred_element_type=jnp.float32)
        hr = h.reshape(BN, 8, N_HIDDEN)
        el = jnp.sum(hr * al_ref[...][None, :, :], axis=-1)
        er = jnp.sum(hr * ar_ref[...][None, :, :], axis=-1)
        el_ref[...] = el
        er_ref[...] = er
        ht = jnp.transpose(hr, (1, 0, 2))
        ones = jnp.ones((8, BN, 1), jnp.float32)
        zeros = jnp.zeros((8, BN, DP - N_HIDDEN - 1), jnp.float32)
        tab_ref[...] = jnp.concatenate([ht, ones, zeros], axis=-1)

    return pl.pallas_call(
        body,
        grid=(NB,),
        in_specs=[
            pl.BlockSpec((BN, IN_FEATS), lambda i: (i, 0)),
            pl.BlockSpec((IN_FEATS, 8 * N_HIDDEN), lambda i: (0, 0)),
            pl.BlockSpec((8, N_HIDDEN), lambda i: (0, 0)),
            pl.BlockSpec((8, N_HIDDEN), lambda i: (0, 0)),
        ],
        out_specs=[
            pl.BlockSpec((8, BN, DP), lambda i: (0, i, 0)),
            pl.BlockSpec((BN, 8), lambda i: (i, 0)),
            pl.BlockSpec((BN, 8), lambda i: (i, 0)),
        ],
        out_shape=[
            jax.ShapeDtypeStruct((8, N, DP), jnp.float32),
            jax.ShapeDtypeStruct((N, 8), jnp.float32),
            jax.ShapeDtypeStruct((N, 8), jnp.float32),
        ],
    )(x, W1, al1, ar1)


def _tc_layer2(acc1, W2, al2, ar2):
    def body(a_ref, w_ref, al_ref, ar_ref, tab_ref, el_ref, er_ref):
        a = a_ref[...][0] + a_ref[...][1]
        s = a[:, :, N_HIDDEN:N_HIDDEN + 1]
        o = a[:, :, 0:64] / (s + 1e-9)
        o = jnp.where(o > 0, o, jnp.exp(o) - 1.0)
        h1 = jnp.transpose(o, (1, 0, 2)).reshape(BN, 8 * N_HIDDEN)
        h2 = jnp.dot(h1, w_ref[...], preferred_element_type=jnp.float32)
        el = jnp.sum(h2 * al_ref[...], axis=-1)
        er = jnp.sum(h2 * ar_ref[...], axis=-1)
        el_ref[...] = el[:, None]
        er_ref[...] = er[:, None]
        ones = jnp.ones((BN, 1), jnp.float32)
        zeros = jnp.zeros((BN, DP - N_CLASSES - 1), jnp.float32)
        tab_ref[...] = jnp.concatenate([h2, ones, zeros], axis=-1)

    return pl.pallas_call(
        body,
        grid=(NB,),
        in_specs=[
            pl.BlockSpec((2, 8, BN, DP), lambda i: (0, 0, i, 0)),
            pl.BlockSpec((8 * N_HIDDEN, N_CLASSES), lambda i: (0, 0)),
            pl.BlockSpec((1, N_CLASSES), lambda i: (0, 0)),
            pl.BlockSpec((1, N_CLASSES), lambda i: (0, 0)),
        ],
        out_specs=[
            pl.BlockSpec((BN, DP), lambda i: (i, 0)),
            pl.BlockSpec((BN, 1), lambda i: (i, 0)),
            pl.BlockSpec((BN, 1), lambda i: (i, 0)),
        ],
        out_shape=[
            jax.ShapeDtypeStruct((N, DP), jnp.float32),
            jax.ShapeDtypeStruct((N, 1), jnp.float32),
            jax.ShapeDtypeStruct((N, 1), jnp.float32),
        ],
    )(acc1, W2, al2, ar2)


def _tc_final(acc2):
    def body(a_ref, out_ref):
        a = a_ref[...][0] + a_ref[...][1]
        out_ref[...] = a[:, 0:N_CLASSES] / (a[:, N_CLASSES:N_CLASSES + 1] + 1e-9)

    return pl.pallas_call(
        body,
        grid=(NB,),
        in_specs=[pl.BlockSpec((2, BN, DP), lambda i: (0, i, 0))],
        out_specs=pl.BlockSpec((BN, N_CLASSES), lambda i: (i, 0)),
        out_shape=jax.ShapeDtypeStruct((N, N_CLASSES), jnp.float32),
    )(acc2)


def _make_sc_edge(num_heads, nsq):
    ew = E_PAD // NW
    npair = ew // (2 * K)

    mesh = plsc.VectorSubcoreMesh(core_axis_name="c", subcore_axis_name="s")

    @functools.partial(
        pl.kernel,
        mesh=mesh,
        compiler_params=pltpu.CompilerParams(needs_layout_passes=False),
        out_type=jax.ShapeDtypeStruct((2 * num_heads * NA, DP), jnp.float32),
        scratch_types=[
            pltpu.VMEM((N,), jnp.float32),
            pltpu.VMEM((N,), jnp.float32),
            pltpu.VMEM((2 * K,), jnp.int32),
            pltpu.VMEM((2 * K,), jnp.int32),
            pltpu.VMEM((K,), jnp.int32),
            pltpu.VMEM((K,), jnp.int32),
            pltpu.VMEM((K,), jnp.int32),
            pltpu.VMEM((K,), jnp.int32),
            pltpu.VMEM((K,), jnp.float32),
            pltpu.VMEM((K,), jnp.float32),
            pltpu.VMEM((K, DP), jnp.float32),
            pltpu.VMEM((K, DP), jnp.float32),
            pltpu.VMEM_SHARED((NA, DP), jnp.float32),
            pltpu.SemaphoreType.DMA,
            pltpu.SemaphoreType.DMA,
            pltpu.SemaphoreType.DMA,
            pltpu.SemaphoreType.DMA,
            pltpu.SemaphoreType.DMA,
        ],
    )
    def sc_kernel(tab_hbm, el_hbm, er_hbm, sd_hbm, out_hbm,
                  el_v, er_v, sd_a, sd_b, gi_a, gi_b, di_a, di_b,
                  w_a, w_b, rows_a, rows_b, acc,
                  gsem_a, gsem_b, csem_a, csem_b, ssem):
        c = lax.axis_index("c")
        s = lax.axis_index("s")
        wid = s * 2 + c
        row0 = s * RPT
        zvec = jnp.zeros((16,), jnp.float32)

        def zero_acc():
            for r in range(K):
                for q in range(DP // 16):
                    rows_a[r, pl.ds(16 * q, 16)] = zvec
            for kk in range(RPT // K):
                pltpu.sync_copy(rows_a, acc.at[pl.ds(row0 + K * kk, K)])
            rem = RPT % K
            if rem:
                pltpu.sync_copy(rows_a.at[pl.ds(0, rem)],
                                acc.at[pl.ds(row0 + (RPT // K) * K, rem)])

        zero_acc()
        plsc.subcore_barrier()

        bufs = [
            (sd_a, gi_a, di_a, w_a, rows_a, gsem_a, csem_a),
            (sd_b, gi_b, di_b, w_b, rows_b, gsem_b, csem_b),
        ]

        cols = [lax.iota(jnp.int32, 16) + 16 * q for q in range(nsq)]

        def head_body(h, _):
            hbase = pl.multiple_of(h * N, 8)
            pltpu.sync_copy(el_hbm.at[pl.ds(hbase, N)], el_v)
            pltpu.sync_copy(er_hbm.at[pl.ds(hbase, N)], er_v)
            hoff = jnp.full((16,), h * N, jnp.int32)

            def pair_body(t2, _):
                pbase = (wid * ew + t2 * 2 * K) * 2

                @pl.when(t2 == 0)
                def _():
                    pltpu.sync_copy(sd_hbm.at[pl.ds(pbase, 2 * K)], sd_a)
                    pltpu.sync_copy(sd_hbm.at[pl.ds(pbase + 2 * K, 2 * K)],
                                    sd_b)
                @pl.when(t2 > 0)
                def _():
                    pltpu.make_async_copy(
                        sd_hbm.at[pl.ds(pbase, 2 * K)], sd_a, ssem).wait()
                    pltpu.make_async_copy(
                        sd_hbm.at[pl.ds(pbase + 2 * K, 2 * K)], sd_b,
                        ssem).wait()

                for sd_v, gi, di, w_v, rows_v, gsem, csem in bufs:
                    @pl.when(t2 > 0)
                    def _():
                        pltpu.make_async_copy(
                            rows_v, acc.at[di], csem).wait()
                    for j in range(K // 16):
                        s16 = sd_v[pl.ds(j * 16, 16)]
                        d16 = sd_v[pl.ds(K + j * 16, 16)]
                        dc = jnp.minimum(d16, N - 1)
                        e16 = (plsc.load_gather(el_v, [s16])
                               + plsc.load_gather(er_v, [dc]))
                        e16 = jnp.where(e16 > 0, e16, NEG_SLOPE * e16)
                        w_v[pl.ds(j * 16, 16)] = jnp.exp(e16)
                        if num_heads > 1:
                            gi[pl.ds(j * 16, 16)] = s16 + hoff
                        else:
                            gi[pl.ds(j * 16, 16)] = s16
                        di[pl.ds(j * 16, 16)] = d16
                    pltpu.async_copy(tab_hbm.at[gi], rows_v, gsem)

                @pl.when(t2 + 1 < npair)
                def _():
                    nbase = pbase + 4 * K
                    pltpu.async_copy(sd_hbm.at[pl.ds(nbase, 2 * K)], sd_a,
                                     ssem)
                    pltpu.async_copy(sd_hbm.at[pl.ds(nbase + 2 * K, 2 * K)],
                                     sd_b, ssem)

                for sd_v, gi, di, w_v, rows_v, gsem, csem in bufs:
                    pltpu.make_async_copy(tab_hbm.at[gi], rows_v, gsem).wait()
                    @plsc.parallel_loop(0, K, step=1, unroll=16)
                    def _(i):
                        ri = jnp.full((16,), i, jnp.int32)
                        wv = plsc.load_gather(w_v, [ri])
                        for q in range(nsq):
                            vals = plsc.load_gather(rows_v, [ri, cols[q]])
                            plsc.store_scatter(rows_v, [ri, cols[q]],
                                               vals * wv)

                    pltpu.async_copy(rows_v, acc.at[di], csem, add=True)
                return 0

            lax.fori_loop(0, npair, pair_body, 0)
            for _, gi, di, w_v, rows_v, gsem, csem in bufs:
                pltpu.make_async_copy(rows_v, acc.at[di], csem).wait()
            plsc.subcore_barrier()
            dbase = pl.multiple_of((c * num_heads + h) * NA + row0, 8)
            pltpu.sync_copy(acc.at[pl.ds(row0, RPT)],
                            out_hbm.at[pl.ds(dbase, RPT)])
            @pl.when(h + 1 < num_heads)
            def _():
                zero_acc()
            plsc.subcore_barrier()
            return 0

        lax.fori_loop(0, num_heads, head_body, 0)

    return sc_kernel


_sc_edge_l1 = _make_sc_edge(8, 5)
_sc_edge_l2 = _make_sc_edge(1, 3)


def kernel(x, edge_index, W1, al1, ar1, W2, al2, ar2):
    npad = E_PAD - E
    pad_src = jnp.arange(npad, dtype=jnp.int32) % N
    pad_dst = N + jnp.arange(npad, dtype=jnp.int32) % N_JUNK
    src = jnp.concatenate([edge_index[0], pad_src])
    dst = jnp.concatenate([edge_index[1], pad_dst])
    sd = jnp.stack([src.reshape(-1, K), dst.reshape(-1, K)], 1).reshape(-1)

    tab1, el1, er1 = _tc_layer1(x, W1, al1, ar1)
    acc1 = _sc_edge_l1(tab1.reshape(8 * N, DP), el1.T.reshape(8 * N),
                       er1.T.reshape(8 * N), sd)
    tab2, el2, er2 = _tc_layer2(acc1.reshape(2, 8, NA, DP), W2, al2, ar2)
    acc2 = _sc_edge_l2(tab2, el2.reshape(N), er2.reshape(N), sd)
    return _tc_final(acc2.reshape(2, NA, DP))


# speedup vs baseline: 31.1887x; 1.0145x over previous
import functools

import jax
import jax.numpy as jnp
from jax import lax
from jax.experimental import pallas as pl
from jax.experimental.pallas import tpu as pltpu
from jax.experimental.pallas import tpu_sc as plsc

N = 10000
E = 320000
IN_FEATS = 128
N_HIDDEN = 64
HEADS = (8, 1)
N_CLASSES = 41
NEG_SLOPE = 0.2

NW = 32
K = 96
E_PAD = 331776
N_JUNK = 112
NA = N + N_JUNK
RPT = NA // 16
NB = 25
BN = N // NB
DP = 128


def _tc_layer1(x, W1, al1, ar1):
    def body(x_ref, w_ref, al_ref, ar_ref, tab_ref, el_ref, er_ref):
        h = jnp.dot(x_ref[...], w_ref[...], preferred_element_type=jnp.float32)
        hr = h.reshape(BN, 8, N_HIDDEN)
        el = jnp.sum(hr * al_ref[...][None, :, :], axis=-1)
        er = jnp.sum(hr * ar_ref[...][None, :, :], axis=-1)
        el_ref[...] = el
        er_ref[...] = er
        ht = jnp.transpose(hr, (1, 0, 2))
        ones = jnp.ones((8, BN, 1), jnp.float32)
        zeros = jnp.zeros((8, BN, DP - N_HIDDEN - 1), jnp.float32)
        tab_ref[...] = jnp.concatenate([ht, ones, zeros], axis=-1)

    return pl.pallas_call(
        body,
        grid=(NB,),
        in_specs=[
            pl.BlockSpec((BN, IN_FEATS), lambda i: (i, 0)),
            pl.BlockSpec((IN_FEATS, 8 * N_HIDDEN), lambda i: (0, 0)),
            pl.BlockSpec((8, N_HIDDEN), lambda i: (0, 0)),
            pl.BlockSpec((8, N_HIDDEN), lambda i: (0, 0)),
        ],
        out_specs=[
            pl.BlockSpec((8, BN, DP), lambda i: (0, i, 0)),
            pl.BlockSpec((BN, 8), lambda i: (i, 0)),
            pl.BlockSpec((BN, 8), lambda i: (i, 0)),
        ],
        out_shape=[
            jax.ShapeDtypeStruct((8, N, DP), jnp.float32),
            jax.ShapeDtypeStruct((N, 8), jnp.float32),
            jax.ShapeDtypeStruct((N, 8), jnp.float32),
        ],
    )(x, W1, al1, ar1)


def _tc_layer2(acc1, W2, al2, ar2):
    def body(a_ref, w_ref, al_ref, ar_ref, tab_ref, el_ref, er_ref):
        a = a_ref[...][0] + a_ref[...][1]
        s = a[:, :, N_HIDDEN:N_HIDDEN + 1]
        o = a[:, :, 0:64] / (s + 1e-9)
        o = jnp.where(o > 0, o, jnp.exp(o) - 1.0)
        h1 = jnp.transpose(o, (1, 0, 2)).reshape(BN, 8 * N_HIDDEN)
        h2 = jnp.dot(h1, w_ref[...], preferred_element_type=jnp.float32)
        el = jnp.sum(h2 * al_ref[...], axis=-1)
        er = jnp.sum(h2 * ar_ref[...], axis=-1)
        el_ref[...] = el[:, None]
        er_ref[...] = er[:, None]
        ones = jnp.ones((BN, 1), jnp.float32)
        zeros = jnp.zeros((BN, DP - N_CLASSES - 1), jnp.float32)
        tab_ref[...] = jnp.concatenate([h2, ones, zeros], axis=-1)

    return pl.pallas_call(
        body,
        grid=(NB,),
        in_specs=[
            pl.BlockSpec((2, 8, BN, DP), lambda i: (0, 0, i, 0)),
            pl.BlockSpec((8 * N_HIDDEN, N_CLASSES), lambda i: (0, 0)),
            pl.BlockSpec((1, N_CLASSES), lambda i: (0, 0)),
            pl.BlockSpec((1, N_CLASSES), lambda i: (0, 0)),
        ],
        out_specs=[
            pl.BlockSpec((BN, DP), lambda i: (i, 0)),
            pl.BlockSpec((BN, 1), lambda i: (i, 0)),
            pl.BlockSpec((BN, 1), lambda i: (i, 0)),
        ],
        out_shape=[
            jax.ShapeDtypeStruct((N, DP), jnp.float32),
            jax.ShapeDtypeStruct((N, 1), jnp.float32),
            jax.ShapeDtypeStruct((N, 1), jnp.float32),
        ],
    )(acc1, W2, al2, ar2)


def _tc_final(acc2):
    def body(a_ref, out_ref):
        a = a_ref[...][0] + a_ref[...][1]
        out_ref[...] = a[:, 0:N_CLASSES] / (a[:, N_CLASSES:N_CLASSES + 1] + 1e-9)

    return pl.pallas_call(
        body,
        grid=(NB,),
        in_specs=[pl.BlockSpec((2, BN, DP), lambda i: (0, i, 0))],
        out_specs=pl.BlockSpec((BN, N_CLASSES), lambda i: (i, 0)),
        out_shape=jax.ShapeDtypeStruct((N, N_CLASSES), jnp.float32),
    )(acc2)


def _make_sc_edge(num_heads, nsq):
    ew = E_PAD // NW
    npair = ew // (2 * K)

    mesh = plsc.VectorSubcoreMesh(core_axis_name="c", subcore_axis_name="s")

    @functools.partial(
        pl.kernel,
        mesh=mesh,
        compiler_params=pltpu.CompilerParams(needs_layout_passes=False),
        out_type=jax.ShapeDtypeStruct((2 * num_heads * NA, DP), jnp.float32),
        scratch_types=[
            pltpu.VMEM((N,), jnp.float32),
            pltpu.VMEM((N,), jnp.float32),
            pltpu.VMEM((2 * K,), jnp.int32),
            pltpu.VMEM((2 * K,), jnp.int32),
            pltpu.VMEM((K,), jnp.int32),
            pltpu.VMEM((K,), jnp.int32),
            pltpu.VMEM((K,), jnp.int32),
            pltpu.VMEM((K,), jnp.int32),
            pltpu.VMEM((K,), jnp.float32),
            pltpu.VMEM((K,), jnp.float32),
            pltpu.VMEM((K, DP), jnp.float32),
            pltpu.VMEM((K, DP), jnp.float32),
            pltpu.VMEM_SHARED((NA, DP), jnp.float32),
            pltpu.SemaphoreType.DMA,
            pltpu.SemaphoreType.DMA,
            pltpu.SemaphoreType.DMA,
            pltpu.SemaphoreType.DMA,
            pltpu.SemaphoreType.DMA,
        ],
    )
    def sc_kernel(tab_hbm, el_hbm, er_hbm, sd_hbm, out_hbm,
                  el_v, er_v, sd_a, sd_b, gi_a, gi_b, di_a, di_b,
                  w_a, w_b, rows_a, rows_b, acc,
                  gsem_a, gsem_b, csem_a, csem_b, ssem):
        c = lax.axis_index("c")
        s = lax.axis_index("s")
        wid = s * 2 + c
        row0 = s * RPT
        zvec = jnp.zeros((16,), jnp.float32)

        def zero_acc():
            for r in range(K):
                for q in range(DP // 16):
                    rows_a[r, pl.ds(16 * q, 16)] = zvec
            for kk in range(RPT // K):
                pltpu.sync_copy(rows_a, acc.at[pl.ds(row0 + K * kk, K)])
            rem = RPT % K
            if rem:
                pltpu.sync_copy(rows_a.at[pl.ds(0, rem)],
                                acc.at[pl.ds(row0 + (RPT // K) * K, rem)])

        zero_acc()
        plsc.subcore_barrier()

        bufs = [
            (sd_a, gi_a, di_a, w_a, rows_a, gsem_a, csem_a),
            (sd_b, gi_b, di_b, w_b, rows_b, gsem_b, csem_b),
        ]

        cols = [lax.iota(jnp.int32, 16) + 16 * q for q in range(nsq)]

        def head_body(h, _):
            hbase = pl.multiple_of(h * N, 8)
            pltpu.sync_copy(el_hbm.at[pl.ds(hbase, N)], el_v)
            pltpu.sync_copy(er_hbm.at[pl.ds(hbase, N)], er_v)
            hoff = jnp.full((16,), h * N, jnp.int32)

            def pair_body(t2, _):
                pbase = (wid * ew + t2 * 2 * K) * 2

                @pl.when(t2 == 0)
                def _():
                    pltpu.sync_copy(sd_hbm.at[pl.ds(pbase, 2 * K)], sd_a)
                    pltpu.sync_copy(sd_hbm.at[pl.ds(pbase + 2 * K, 2 * K)],
                                    sd_b)
                @pl.when(t2 > 0)
                def _():
                    pltpu.make_async_copy(
                        sd_hbm.at[pl.ds(pbase, 2 * K)], sd_a, ssem).wait()
                    pltpu.make_async_copy(
                        sd_hbm.at[pl.ds(pbase + 2 * K, 2 * K)], sd_b,
                        ssem).wait()

                for sd_v, gi, di, w_v, rows_v, gsem, csem in bufs:
                    @pl.when(t2 > 0)
                    def _():
                        pltpu.make_async_copy(
                            rows_v, acc.at[di], csem).wait()
                    @plsc.parallel_loop(0, K, step=16, unroll=K // 16)
                    def _(jb):
                        s16 = sd_v[pl.ds(jb, 16)]
                        d16 = sd_v[pl.ds(K + jb, 16)]
                        dc = jnp.minimum(d16, N - 1)
                        e16 = (plsc.load_gather(el_v, [s16])
                               + plsc.load_gather(er_v, [dc]))
                        e16 = jnp.where(e16 > 0, e16, NEG_SLOPE * e16)
                        w_v[pl.ds(jb, 16)] = jnp.exp(e16)
                        if num_heads > 1:
                            gi[pl.ds(jb, 16)] = s16 + hoff
                        else:
                            gi[pl.ds(jb, 16)] = s16
                        di[pl.ds(jb, 16)] = d16

                    pltpu.async_copy(tab_hbm.at[gi], rows_v, gsem)

                @pl.when(t2 + 1 < npair)
                def _():
                    nbase = pbase + 4 * K
                    pltpu.async_copy(sd_hbm.at[pl.ds(nbase, 2 * K)], sd_a,
                                     ssem)
                    pltpu.async_copy(sd_hbm.at[pl.ds(nbase + 2 * K, 2 * K)],
                                     sd_b, ssem)

                for sd_v, gi, di, w_v, rows_v, gsem, csem in bufs:
                    pltpu.make_async_copy(tab_hbm.at[gi], rows_v, gsem).wait()
                    @plsc.parallel_loop(0, K, step=1, unroll=16)
                    def _(i):
                        ri = jnp.full((16,), i, jnp.int32)
                        wv = plsc.load_gather(w_v, [ri])
                        for q in range(nsq):
                            vals = plsc.load_gather(rows_v, [ri, cols[q]])
                            plsc.store_scatter(rows_v, [ri, cols[q]],
                                               vals * wv)

                    pltpu.async_copy(rows_v, acc.at[di], csem, add=True)
                return 0

            lax.fori_loop(0, npair, pair_body, 0)
            for _, gi, di, w_v, rows_v, gsem, csem in bufs:
                pltpu.make_async_copy(rows_v, acc.at[di], csem).wait()
            plsc.subcore_barrier()
            dbase = pl.multiple_of((c * num_heads + h) * NA + row0, 8)
            pltpu.sync_copy(acc.at[pl.ds(row0, RPT)],
                            out_hbm.at[pl.ds(dbase, RPT)])
            @pl.when(h + 1 < num_heads)
            def _():
                zero_acc()
            plsc.subcore_barrier()
            return 0

        lax.fori_loop(0, num_heads, head_body, 0)

    return sc_kernel


_sc_edge_l1 = _make_sc_edge(8, 5)
_sc_edge_l2 = _make_sc_edge(1, 3)


def kernel(x, edge_index, W1, al1, ar1, W2, al2, ar2):
    npad = E_PAD - E
    pad_src = jnp.arange(npad, dtype=jnp.int32) % N
    pad_dst = N + jnp.arange(npad, dtype=jnp.int32) % N_JUNK
    src = jnp.concatenate([edge_index[0], pad_src])
    dst = jnp.concatenate([edge_index[1], pad_dst])
    sd = jnp.stack([src.reshape(-1, K), dst.reshape(-1, K)], 1).reshape(-1)

    tab1, el1, er1 = _tc_layer1(x, W1, al1, ar1)
    acc1 = _sc_edge_l1(tab1.reshape(8 * N, DP), el1.T.reshape(8 * N),
                       er1.T.reshape(8 * N), sd)
    tab2, el2, er2 = _tc_layer2(acc1.reshape(2, 8, NA, DP), W2, al2, ar2)
    acc2 = _sc_edge_l2(tab2, el2.reshape(N), er2.reshape(N), sd)
    return _tc_final(acc2.reshape(2, NA, DP))
